# baseline (speedup 1.0000x reference)
import jax
import jax.numpy as jnp
from jax import lax
from jax.experimental import pallas as pl
from jax.experimental.pallas import tpu as pltpu
from jax.experimental.pallas import tpu_sc as plsc

_NC = 2
_NS = 16
_L = 16

_CH = 256
_NCHUNK = 8
_PW = _CH * _NCHUNK


def _split_tc(d_ref, x_ref, y_ref):
    x_ref[...] = d_ref[..., 0]
    y_ref[...] = d_ref[..., 1]


def _bcast_lane(vec, p):
    sel = jnp.full((_L, 1), p, jnp.int32)
    return lax.gather(
        vec,
        sel,
        lax.GatherDimensionNumbers(
            offset_dims=(),
            collapsed_slice_dims=(0,),
            start_index_map=(0,),
        ),
        slice_sizes=(1,),
        mode=lax.GatherScatterMode.PROMISE_IN_BOUNDS,
    )


def _tent_body(x_hbm, y_hbm, samp_hbm, out_hbm, x_v, y_v, samp_v,
               buf0, buf1, sems):
    wid = lax.axis_index("s") * _NC + lax.axis_index("c")
    nd = wid // 2
    half = wid % 2
    base = half * _PW

    pltpu.sync_copy(x_hbm.at[nd, pl.ds(base, _PW)], x_v)
    pltpu.sync_copy(y_hbm.at[nd, pl.ds(base, _PW)], y_v)
    pltpu.sync_copy(samp_hbm, samp_v)
    s_vregs = [samp_v[pl.ds(_L * k, _L)] for k in range(4)]
    bufs = (buf0, buf1)

    def chunk_compute(c, buf):
        @pl.loop(0, _CH // _L)
        def _group(g):
            xv = x_v[pl.ds(c * _CH + g * _L, _L)]
            yv = y_v[pl.ds(c * _CH + g * _L, _L)]
            for p in range(_L):
                xb = _bcast_lane(xv, p)
                yb = _bcast_lane(yv, p)
                r = g * _L + p
                for k in range(4):
                    u = yb - s_vregs[k]
                    v = s_vregs[k] - xb
                    buf[r, pl.ds(_L * k, _L)] = jnp.maximum(
                        jnp.minimum(u, v), 0.0
                    )

    handles = []
    for c in range(_NCHUNK):
        b = c % 2
        if c >= 2:
            handles[c - 2].wait()
        chunk_compute(c, bufs[b])
        handles.append(
            pltpu.async_copy(
                bufs[b],
                out_hbm.at[nd, pl.ds(base + c * _CH, _CH), :],
                sems.at[b],
            )
        )
    handles[-2].wait()
    handles[-1].wait()


def kernel(diagrams, samples):
    n, P, _ = diagrams.shape
    S = samples.shape[0]
    xp, yp = pl.pallas_call(
        _split_tc,
        grid=(2,),
        in_specs=[pl.BlockSpec((n // 2, P, 2), lambda i: (i, 0, 0))],
        out_specs=[
            pl.BlockSpec((n // 2, P), lambda i: (i, 0)),
            pl.BlockSpec((n // 2, P), lambda i: (i, 0)),
        ],
        out_shape=[
            jax.ShapeDtypeStruct((n, P), jnp.float32),
            jax.ShapeDtypeStruct((n, P), jnp.float32),
        ],
    )(diagrams)
    fn = pl.kernel(
        _tent_body,
        out_type=jax.ShapeDtypeStruct((n, P, S), jnp.float32),
        mesh=plsc.VectorSubcoreMesh(core_axis_name="c", subcore_axis_name="s"),
        scratch_types=[
            pltpu.VMEM((_PW,), jnp.float32),
            pltpu.VMEM((_PW,), jnp.float32),
            pltpu.VMEM((S,), jnp.float32),
            pltpu.VMEM((_CH, S), jnp.float32),
            pltpu.VMEM((_CH, S), jnp.float32),
            pltpu.SemaphoreType.DMA((2,)),
        ],
    )
    return fn(xp, yp, samples)
